# baseline (speedup 1.0000x reference)
import jax
import jax.numpy as jnp
from jax import lax
from jax.experimental import pallas as pl
from jax.experimental.pallas import tpu as pltpu
from jax.experimental.pallas import tpu_sc as plsc

N, K, D = 32768, 1024, 64
NCHUNK = 4
CH_TOK = N // NCHUNK
ROWS_PER_BLOCK = 1024
BLOCKS_PER_CH = CH_TOK // ROWS_PER_BLOCK
NW = 32
B_PER_W = CH_TOK // NW
IDX_CHUNK = 128
CHUNKS = B_PER_W // IDX_CHUNK
WPB = ROWS_PER_BLOCK // B_PER_W


def _argmax_body(x_ref, idx_ref):
    xb = x_ref[...]
    m = jnp.max(xb, axis=-1, keepdims=True)
    col = lax.broadcasted_iota(jnp.int32, xb.shape, 1)
    cand = jnp.where(xb == m, col, K)
    idx_ref[...] = jnp.min(cand, axis=-1).reshape(WPB, CHUNKS, IDX_CHUNK)


def _tc_argmax_chunk(x, c):
    return pl.pallas_call(
        _argmax_body,
        grid=(BLOCKS_PER_CH,),
        in_specs=[
            pl.BlockSpec((ROWS_PER_BLOCK, K), lambda i, c=c: (c * BLOCKS_PER_CH + i, 0))
        ],
        out_specs=pl.BlockSpec((WPB, CHUNKS, IDX_CHUNK), lambda i: (i, 0, 0)),
        out_shape=jax.ShapeDtypeStruct((NW, CHUNKS, IDX_CHUNK), jnp.int32),
    )(x)


def _sc_gather_body(table_hbm, idx_hbm, out_hbm, idx_v, rows_v, sem):
    wid = lax.axis_index("s") * 2 + lax.axis_index("c")
    pltpu.sync_copy(idx_hbm.at[wid], idx_v)
    for j in range(CHUNKS):
        pltpu.async_copy(
            table_hbm.at[idx_v.at[j]],
            rows_v.at[pl.ds(j * IDX_CHUNK, IDX_CHUNK)],
            sem,
        ).wait()
    pltpu.sync_copy(rows_v, out_hbm.at[pl.ds(wid * B_PER_W, B_PER_W)])


def _sc_gather(table, idx3):
    mesh = plsc.VectorSubcoreMesh(core_axis_name="c", subcore_axis_name="s")
    run = pl.kernel(
        _sc_gather_body,
        out_type=jax.ShapeDtypeStruct((CH_TOK, D), jnp.float32),
        mesh=mesh,
        scratch_types=[
            pltpu.VMEM((CHUNKS, IDX_CHUNK), jnp.int32),
            pltpu.VMEM((B_PER_W, D), jnp.float32),
            pltpu.SemaphoreType.DMA,
        ],
        compiler_params=pltpu.CompilerParams(use_tc_tiling_on_sc=False),
    )
    return run(table, idx3)


def kernel(x, W):
    table = jnp.transpose(W)
    outs = []
    for c in range(NCHUNK):
        idx3 = _tc_argmax_chunk(x, c)
        outs.append(_sc_gather(table, idx3))
    return jnp.concatenate(outs, axis=0)


# speedup vs baseline: 1.1369x; 1.1369x over previous
import jax
import jax.numpy as jnp
from jax import lax
from jax.experimental import pallas as pl
from jax.experimental.pallas import tpu as pltpu
from jax.experimental.pallas import tpu_sc as plsc

N, K, D = 32768, 1024, 64
ROWS_PER_BLOCK = 1024
NUM_BLOCKS = N // ROWS_PER_BLOCK
NW = 32
B_PER_W = N // NW
PAIRS_PER_W = B_PER_W // 2
IDX_CHUNK = 128
CHUNKS = PAIRS_PER_W // IDX_CHUNK


def _argmax_body(x_ref, idx_ref):
    xb = x_ref[...]
    m = jnp.max(xb, axis=-1, keepdims=True)
    col = lax.broadcasted_iota(jnp.int32, xb.shape, 1)
    cand = jnp.where(xb == m, col, K)
    am = jnp.min(cand, axis=-1)
    am2 = am.reshape(PAIRS_PER_W, 2)
    idx_ref[0, 0] = am2[:, 0].reshape(CHUNKS, IDX_CHUNK)
    idx_ref[0, 1] = am2[:, 1].reshape(CHUNKS, IDX_CHUNK)


def _tc_argmax(x):
    return pl.pallas_call(
        _argmax_body,
        grid=(NUM_BLOCKS,),
        in_specs=[pl.BlockSpec((ROWS_PER_BLOCK, K), lambda i: (i, 0))],
        out_specs=pl.BlockSpec((1, 2, CHUNKS, IDX_CHUNK), lambda i: (i, 0, 0, 0)),
        out_shape=jax.ShapeDtypeStruct((NW, 2, CHUNKS, IDX_CHUNK), jnp.int32),
    )(x)


def _sc_gather_body(table_hbm, idx_hbm, out_hbm, idx_v, even_v, odd_v, sem):
    wid = lax.axis_index("s") * 2 + lax.axis_index("c")
    pltpu.sync_copy(idx_hbm.at[wid], idx_v)
    for p, buf in ((0, even_v), (1, odd_v)):
        for j in range(CHUNKS):
            pltpu.async_copy(
                table_hbm.at[idx_v.at[p, j]],
                buf.at[pl.ds(j * IDX_CHUNK, IDX_CHUNK)],
                sem,
            ).wait()
    base = wid * PAIRS_PER_W
    pltpu.sync_copy(even_v, out_hbm.at[pl.ds(base, PAIRS_PER_W), pl.ds(0, D)])
    pltpu.sync_copy(odd_v, out_hbm.at[pl.ds(base, PAIRS_PER_W), pl.ds(D, D)])


def _sc_gather(table, idx4):
    mesh = plsc.VectorSubcoreMesh(core_axis_name="c", subcore_axis_name="s")
    run = pl.kernel(
        _sc_gather_body,
        out_type=jax.ShapeDtypeStruct((N // 2, 2 * D), jnp.float32),
        mesh=mesh,
        scratch_types=[
            pltpu.VMEM((2, CHUNKS, IDX_CHUNK), jnp.int32),
            pltpu.VMEM((PAIRS_PER_W, D), jnp.float32),
            pltpu.VMEM((PAIRS_PER_W, D), jnp.float32),
            pltpu.SemaphoreType.DMA,
        ],
        compiler_params=pltpu.CompilerParams(use_tc_tiling_on_sc=False),
    )
    return run(table, idx4)


def kernel(x, W):
    table = jnp.transpose(W)
    idx4 = _tc_argmax(x)
    out2 = _sc_gather(table, idx4)
    return out2.reshape(N, D)
